# baseline (speedup 1.0000x reference)
import functools

import jax
import jax.numpy as jnp
from jax import lax
from jax.experimental import pallas as pl
from jax.experimental.pallas import tpu as pltpu
from jax.experimental.pallas import tpu_sc as plsc

N_IN = 32768
KSEL = 1024
BATCH = 4096
ROWS = 256
COLS = 128

NC, NS, LANES = 2, 16, 16
NW = NC * NS
ROWS_PER_W = BATCH // NW
CHUNK = 128
NCHUNK = KSEL // CHUNK


def _select_body(logits_ref, sel_ref, posm_ref):
    x = logits_ref[...]
    bits = lax.bitcast_convert_type(x, jnp.int32)
    key = bits ^ jnp.right_shift(bits, 31).astype(jnp.int32) & jnp.int32(0x7FFFFFFF)

    cnt_pos = jnp.sum((key >= 0).astype(jnp.int32))
    p0 = jnp.where(cnt_pos >= KSEL, jnp.int32(0), jnp.int32(-2147483648))
    kk0 = jnp.where(cnt_pos >= KSEL, jnp.int32(KSEL), jnp.int32(KSEL) - cnt_pos)
    mkn0 = jnp.int32(-2147483648)

    def bit_step(i, carry):
        p, kk, mkn = carry
        b = jnp.int32(30) - i
        bit = jnp.left_shift(jnp.int32(1), b)
        test = p | bit
        m = mkn | bit
        cnt = jnp.sum(((key & m) == test).astype(jnp.int32))
        take = cnt >= kk
        p = jnp.where(take, test, p)
        kk = jnp.where(take, kk, kk - cnt)
        return p, kk, m

    T, _, _ = lax.fori_loop(0, 31, bit_step, (p0, kk0, mkn0))

    c_gt = jnp.sum((key > T).astype(jnp.int32))
    r = jnp.int32(KSEL) - c_gt

    gi = (lax.broadcasted_iota(jnp.int32, (ROWS, COLS), 0) * COLS
          + lax.broadcasted_iota(jnp.int32, (ROWS, COLS), 1))
    eq = key == T

    def bs_step(_, carry):
        lo, hi = carry
        mid = (lo + hi) >> 1
        f = jnp.sum((eq & (gi < mid)).astype(jnp.int32))
        ge = f >= r
        return jnp.where(ge, lo, mid), jnp.where(ge, mid, hi)

    _, c = lax.fori_loop(0, 15, bs_step, (jnp.int32(0), jnp.int32(N_IN)))

    mask = (key > T) | (eq & (gi < c))
    maskf = mask.astype(jnp.float32)
    sel_ref[...] = maskf

    cj = lax.broadcasted_iota(jnp.int32, (COLS, COLS), 0)
    ck = lax.broadcasted_iota(jnp.int32, (COLS, COLS), 1)
    su = (cj < ck).astype(jnp.float32)
    pref = jnp.dot(maskf, su, preferred_element_type=jnp.float32)
    ri = lax.broadcasted_iota(jnp.int32, (ROWS, ROWS), 0)
    rk = lax.broadcasted_iota(jnp.int32, (ROWS, ROWS), 1)
    sl = (rk < ri).astype(jnp.float32)
    rowsum = jnp.sum(maskf, axis=1, keepdims=True)
    rowoff = jnp.dot(sl, jnp.broadcast_to(rowsum, (ROWS, COLS)),
                     preferred_element_type=jnp.float32)
    pos = (rowoff + pref).astype(jnp.int32)
    posm_ref[...] = jnp.where(mask, pos, jnp.int32(-1))


@jax.jit
def _tc_select(gate_logits):
    sel2d, posm = pl.pallas_call(
        _select_body,
        out_shape=(
            jax.ShapeDtypeStruct((ROWS, COLS), jnp.float32),
            jax.ShapeDtypeStruct((ROWS, COLS), jnp.int32),
        ),
    )(gate_logits.reshape(ROWS, COLS))
    return sel2d.reshape(N_IN), posm.reshape(N_IN)


def _sc_body(feat_hbm, posm_hbm, out_hbm,
             posm_v, idx_v, idxrow_v, rowbuf_v, sem_g, sem_o):
    wid = lax.axis_index("s") * NC + lax.axis_index("c")
    base_row = wid * ROWS_PER_W

    pltpu.sync_copy(posm_hbm, posm_v)

    def scat(j, carry):
        v = posm_v[pl.ds(j * LANES, LANES)]
        m = v >= 0
        gidx = lax.iota(jnp.int32, LANES) + j * LANES
        plsc.store_scatter(idx_v, [v], gidx, mask=m)
        return carry

    lax.fori_loop(0, N_IN // LANES, scat, 0)

    def row_loop(r, carry):
        b = base_row + r
        rbase = b * N_IN

        def mkidx(j, carry2):
            idxrow_v[pl.ds(j * LANES, LANES)] = (
                idx_v[pl.ds(j * LANES, LANES)] + rbase)
            return carry2

        lax.fori_loop(0, KSEL // LANES, mkidx, 0)

        cps = []
        for k in range(NCHUNK):
            cps.append(pltpu.async_copy(
                feat_hbm.at[idxrow_v.at[pl.ds(k * CHUNK, CHUNK)]],
                rowbuf_v.at[pl.ds(k * CHUNK, CHUNK)],
                sem_g))
        for cp in cps:
            cp.wait()
        pltpu.sync_copy(rowbuf_v, out_hbm.at[b])
        return carry

    lax.fori_loop(0, ROWS_PER_W, row_loop, 0)


@functools.cache
def _sc_gather():
    return functools.partial(
        pl.kernel,
        out_type=jax.ShapeDtypeStruct((BATCH, KSEL), jnp.float32),
        mesh=plsc.VectorSubcoreMesh(core_axis_name="c", subcore_axis_name="s"),
        compiler_params=pltpu.CompilerParams(needs_layout_passes=False),
        scratch_types=[
            pltpu.VMEM((N_IN,), jnp.int32),
            pltpu.VMEM((KSEL,), jnp.int32),
            pltpu.VMEM((KSEL,), jnp.int32),
            pltpu.VMEM((KSEL,), jnp.float32),
            pltpu.SemaphoreType.DMA,
            pltpu.SemaphoreType.DMA,
        ],
    )(_sc_body)


def kernel(features, gate_logits):
    sel_weights, posm = _tc_select(gate_logits)
    selected = _sc_gather()(features.reshape(-1), posm)
    return selected, sel_weights


# speedup vs baseline: 2.1445x; 2.1445x over previous
import functools

import jax
import jax.numpy as jnp
from jax import lax
from jax.experimental import pallas as pl
from jax.experimental.pallas import tpu as pltpu
from jax.experimental.pallas import tpu_sc as plsc

N_IN = 32768
KSEL = 1024
BATCH = 4096
ROWS = 256
COLS = 128

NC, NS, LANES = 2, 16, 16
NW = NC * NS
ROWS_PER_W = BATCH // NW
CHUNK = 128
NCHUNK = KSEL // CHUNK


def _select_body(logits_ref, sel_ref, posm_ref):
    x = logits_ref[...]
    bits = lax.bitcast_convert_type(x, jnp.int32)
    key = bits ^ jnp.right_shift(bits, 31).astype(jnp.int32) & jnp.int32(0x7FFFFFFF)

    cnt_pos = jnp.sum((key >= 0).astype(jnp.int32))
    p0 = jnp.where(cnt_pos >= KSEL, jnp.int32(0), jnp.int32(-2147483648))
    kk0 = jnp.where(cnt_pos >= KSEL, jnp.int32(KSEL), jnp.int32(KSEL) - cnt_pos)
    mkn0 = jnp.int32(-2147483648)

    def bit_step(i, carry):
        p, kk, mkn = carry
        b = jnp.int32(30) - i
        bit = jnp.left_shift(jnp.int32(1), b)
        test = p | bit
        m = mkn | bit
        cnt = jnp.sum(((key & m) == test).astype(jnp.int32))
        take = cnt >= kk
        p = jnp.where(take, test, p)
        kk = jnp.where(take, kk, kk - cnt)
        return p, kk, m

    T, _, _ = lax.fori_loop(0, 31, bit_step, (p0, kk0, mkn0))

    c_gt = jnp.sum((key > T).astype(jnp.int32))
    r = jnp.int32(KSEL) - c_gt

    gi = (lax.broadcasted_iota(jnp.int32, (ROWS, COLS), 0) * COLS
          + lax.broadcasted_iota(jnp.int32, (ROWS, COLS), 1))
    eq = key == T

    def bs_step(_, carry):
        lo, hi = carry
        mid = (lo + hi) >> 1
        f = jnp.sum((eq & (gi < mid)).astype(jnp.int32))
        ge = f >= r
        return jnp.where(ge, lo, mid), jnp.where(ge, mid, hi)

    _, c = lax.fori_loop(0, 15, bs_step, (jnp.int32(0), jnp.int32(N_IN)))

    mask = (key > T) | (eq & (gi < c))
    maskf = mask.astype(jnp.float32)
    sel_ref[...] = maskf

    cj = lax.broadcasted_iota(jnp.int32, (COLS, COLS), 0)
    ck = lax.broadcasted_iota(jnp.int32, (COLS, COLS), 1)
    su = (cj < ck).astype(jnp.float32)
    pref = jnp.dot(maskf, su, preferred_element_type=jnp.float32)
    ri = lax.broadcasted_iota(jnp.int32, (ROWS, ROWS), 0)
    rk = lax.broadcasted_iota(jnp.int32, (ROWS, ROWS), 1)
    sl = (rk < ri).astype(jnp.float32)
    rowsum = jnp.sum(maskf, axis=1, keepdims=True)
    rowoff = jnp.dot(sl, jnp.broadcast_to(rowsum, (ROWS, COLS)),
                     preferred_element_type=jnp.float32)
    pos = (rowoff + pref).astype(jnp.int32)
    posm_ref[...] = jnp.where(mask, pos, jnp.int32(-1))


@jax.jit
def _tc_select(gate_logits):
    sel2d, posm = pl.pallas_call(
        _select_body,
        out_shape=(
            jax.ShapeDtypeStruct((ROWS, COLS), jnp.float32),
            jax.ShapeDtypeStruct((ROWS, COLS), jnp.int32),
        ),
    )(gate_logits.reshape(ROWS, COLS))
    return sel2d.reshape(N_IN), posm.reshape(N_IN)


def _sc_body(feat_hbm, posm_hbm, out_hbm,
             posm_v, idx_v, idxrow_v, rowbuf_v, sem_g, sem_o):
    wid = lax.axis_index("s") * NC + lax.axis_index("c")
    base_row = wid * ROWS_PER_W

    pltpu.sync_copy(posm_hbm, posm_v)

    def scat(j, carry):
        v = posm_v[pl.ds(j * LANES, LANES)]
        m = v >= 0
        gidx = lax.iota(jnp.int32, LANES) + j * LANES
        plsc.store_scatter(idx_v, [v], gidx, mask=m)
        return carry

    lax.fori_loop(0, N_IN // LANES, scat, 0)

    def phys(j, carry):
        v = idx_v[pl.ds(j * LANES, LANES)]
        idx_v[pl.ds(j * LANES, LANES)] = (
            jnp.left_shift(jnp.right_shift(v, 7), 10) + (v & 127))
        return carry

    lax.fori_loop(0, KSEL // LANES, phys, 0)

    def row_loop(r, carry):
        b = base_row + r
        rbase = jnp.left_shift(jnp.right_shift(b, 3), 18) + (
            jnp.left_shift(b & 7, 7))

        def mkidx(j, carry2):
            idxrow_v[pl.ds(j * LANES, LANES)] = (
                idx_v[pl.ds(j * LANES, LANES)] + rbase)
            return carry2

        lax.fori_loop(0, KSEL // LANES, mkidx, 0)

        cps = []
        for k in range(NCHUNK):
            cps.append(pltpu.async_copy(
                feat_hbm.at[idxrow_v.at[pl.ds(k * CHUNK, CHUNK)]],
                rowbuf_v.at[pl.ds(k * CHUNK, CHUNK)],
                sem_g))
        for cp in cps:
            cp.wait()
        pltpu.sync_copy(rowbuf_v, out_hbm.at[b])
        return carry

    lax.fori_loop(0, ROWS_PER_W, row_loop, 0)


@functools.cache
def _sc_gather():
    return functools.partial(
        pl.kernel,
        out_type=jax.ShapeDtypeStruct((BATCH, KSEL), jnp.float32),
        mesh=plsc.VectorSubcoreMesh(core_axis_name="c", subcore_axis_name="s"),
        compiler_params=pltpu.CompilerParams(needs_layout_passes=False),
        scratch_types=[
            pltpu.VMEM((N_IN,), jnp.int32),
            pltpu.VMEM((KSEL,), jnp.int32),
            pltpu.VMEM((KSEL,), jnp.int32),
            pltpu.VMEM((KSEL,), jnp.float32),
            pltpu.SemaphoreType.DMA,
            pltpu.SemaphoreType.DMA,
        ],
    )(_sc_body)


def kernel(features, gate_logits):
    sel_weights, posm = _tc_select(gate_logits)
    feats_lin = (features.reshape(BATCH // 8, 8, N_IN // 128, 128)
                 .transpose(0, 2, 1, 3).reshape(-1))
    selected = _sc_gather()(feats_lin, posm)
    return selected, sel_weights


# speedup vs baseline: 3.1729x; 1.4795x over previous
import functools

import jax
import jax.numpy as jnp
from jax import lax
from jax.experimental import pallas as pl
from jax.experimental.pallas import tpu as pltpu
from jax.experimental.pallas import tpu_sc as plsc

N_IN = 32768
KSEL = 1024
BATCH = 4096
ROWS = 256
COLS = 128

NC, NS, LANES = 2, 16, 16
NW = NC * NS
ROWS_PER_W = BATCH // NW
CHUNK = 128
NCHUNK = KSEL // CHUNK


def _select_body(logits_ref, sel_ref, posm_ref):
    x = logits_ref[...]
    bits = lax.bitcast_convert_type(x, jnp.int32)
    key = bits ^ jnp.right_shift(bits, 31).astype(jnp.int32) & jnp.int32(0x7FFFFFFF)

    cnt_pos = jnp.sum((key >= 0).astype(jnp.int32))
    p0 = jnp.where(cnt_pos >= KSEL, jnp.int32(0), jnp.int32(-2147483648))
    kk0 = jnp.where(cnt_pos >= KSEL, jnp.int32(KSEL), jnp.int32(KSEL) - cnt_pos)
    mkn0 = jnp.int32(-2147483648)

    def bit_step(i, carry):
        p, kk, mkn = carry
        b = jnp.int32(30) - i
        bit = jnp.left_shift(jnp.int32(1), b)
        test = p | bit
        m = mkn | bit
        cnt = jnp.sum(((key & m) == test).astype(jnp.int32))
        take = cnt >= kk
        p = jnp.where(take, test, p)
        kk = jnp.where(take, kk, kk - cnt)
        return p, kk, m

    T, _, _ = lax.fori_loop(0, 31, bit_step, (p0, kk0, mkn0))

    c_gt = jnp.sum((key > T).astype(jnp.int32))
    r = jnp.int32(KSEL) - c_gt

    gi = (lax.broadcasted_iota(jnp.int32, (ROWS, COLS), 0) * COLS
          + lax.broadcasted_iota(jnp.int32, (ROWS, COLS), 1))
    eq = key == T

    def bs_step(_, carry):
        lo, hi = carry
        mid = (lo + hi) >> 1
        f = jnp.sum((eq & (gi < mid)).astype(jnp.int32))
        ge = f >= r
        return jnp.where(ge, lo, mid), jnp.where(ge, mid, hi)

    _, c = lax.fori_loop(0, 15, bs_step, (jnp.int32(0), jnp.int32(N_IN)))

    mask = (key > T) | (eq & (gi < c))
    maskf = mask.astype(jnp.float32)
    sel_ref[...] = maskf

    cj = lax.broadcasted_iota(jnp.int32, (COLS, COLS), 0)
    ck = lax.broadcasted_iota(jnp.int32, (COLS, COLS), 1)
    su = (cj < ck).astype(jnp.float32)
    pref = jnp.dot(maskf, su, preferred_element_type=jnp.float32)
    ri = lax.broadcasted_iota(jnp.int32, (ROWS, ROWS), 0)
    rk = lax.broadcasted_iota(jnp.int32, (ROWS, ROWS), 1)
    sl = (rk < ri).astype(jnp.float32)
    rowsum = jnp.sum(maskf, axis=1, keepdims=True)
    rowoff = jnp.dot(sl, jnp.broadcast_to(rowsum, (ROWS, COLS)),
                     preferred_element_type=jnp.float32)
    pos = (rowoff + pref).astype(jnp.int32)
    posm_ref[...] = jnp.where(mask, pos, jnp.int32(-1))


@jax.jit
def _tc_select(gate_logits):
    sel2d, posm = pl.pallas_call(
        _select_body,
        out_shape=(
            jax.ShapeDtypeStruct((ROWS, COLS), jnp.float32),
            jax.ShapeDtypeStruct((ROWS, COLS), jnp.int32),
        ),
    )(gate_logits.reshape(ROWS, COLS))
    return sel2d.reshape(N_IN), posm.reshape(N_IN)


NBUF = 4
UNROLL = 8


def _sc_body(feat_hbm, posm_hbm, out_hbm,
             posm_v, idx_v, idxrow_v, rowbuf_v, sem_g, sem_o):
    wid = lax.axis_index("s") * NC + lax.axis_index("c")
    base_row = wid * ROWS_PER_W

    pltpu.sync_copy(posm_hbm, posm_v)

    def scat(jj, carry):
        for jo in range(UNROLL):
            j = jj * UNROLL + jo
            v = posm_v[pl.ds(j * LANES, LANES)]
            m = v >= 0
            gidx = lax.iota(jnp.int32, LANES) + j * LANES
            poff = jnp.left_shift(jnp.right_shift(gidx, 7), 10) + (gidx & 127)
            plsc.store_scatter(idx_v, [v], poff, mask=m)
        return carry

    lax.fori_loop(0, N_IN // LANES // UNROLL, scat, 0)

    def mkidx(u, b):
        rbase = jnp.left_shift(jnp.right_shift(b, 3), 18) + (
            jnp.left_shift(b & 7, 7))

        def step(jj, carry2):
            for jo in range(UNROLL):
                j = jj * UNROLL + jo
                idxrow_v[u, pl.ds(j * LANES, LANES)] = (
                    idx_v[pl.ds(j * LANES, LANES)] + rbase)
            return carry2

        lax.fori_loop(0, KSEL // LANES // UNROLL, step, 0)

    def group_loop(g, carry):
        cps = {}
        for u in range(NBUF):
            r = g * NBUF + u

            @pl.when(g > 0)
            def _():
                pltpu.make_async_copy(
                    rowbuf_v.at[u], out_hbm.at[0], sem_o[u]).wait()

            mkidx(u, base_row + r)
            cps[u] = [
                pltpu.async_copy(
                    feat_hbm.at[idxrow_v.at[u, pl.ds(k * CHUNK, CHUNK)]],
                    rowbuf_v.at[u, pl.ds(k * CHUNK, CHUNK)],
                    sem_g[u])
                for k in range(NCHUNK)
            ]
        for u in range(NBUF):
            for cp in cps[u]:
                cp.wait()
            pltpu.async_copy(rowbuf_v.at[u], out_hbm.at[base_row + g * NBUF + u],
                             sem_o[u])
        return carry

    lax.fori_loop(0, ROWS_PER_W // NBUF, group_loop, 0)
    for u in range(NBUF):
        pltpu.make_async_copy(rowbuf_v.at[u], out_hbm.at[0], sem_o[u]).wait()


@functools.cache
def _sc_gather():
    return functools.partial(
        pl.kernel,
        out_type=jax.ShapeDtypeStruct((BATCH, KSEL), jnp.float32),
        mesh=plsc.VectorSubcoreMesh(core_axis_name="c", subcore_axis_name="s"),
        compiler_params=pltpu.CompilerParams(needs_layout_passes=False),
        scratch_types=[
            pltpu.VMEM((N_IN,), jnp.int32),
            pltpu.VMEM((KSEL,), jnp.int32),
            pltpu.VMEM((NBUF, KSEL), jnp.int32),
            pltpu.VMEM((NBUF, KSEL), jnp.float32),
            [pltpu.SemaphoreType.DMA] * NBUF,
            [pltpu.SemaphoreType.DMA] * NBUF,
        ],
    )(_sc_body)


def kernel(features, gate_logits):
    sel_weights, posm = _tc_select(gate_logits)
    feats_lin = (features.reshape(BATCH // 8, 8, N_IN // 128, 128)
                 .transpose(0, 2, 1, 3).reshape(-1))
    selected = _sc_gather()(feats_lin, posm)
    return selected, sel_weights


# speedup vs baseline: 3.3561x; 1.0577x over previous
import functools

import jax
import jax.numpy as jnp
from jax import lax
from jax.experimental import pallas as pl
from jax.experimental.pallas import tpu as pltpu
from jax.experimental.pallas import tpu_sc as plsc

N_IN = 32768
KSEL = 1024
BATCH = 4096
ROWS = 256
COLS = 128

NC, NS, LANES = 2, 16, 16
NW = NC * NS
ROWS_PER_W = BATCH // NW
CHUNK = 128
NCHUNK = KSEL // CHUNK


def _select_body(logits_ref, sel_ref, posm_ref):
    x = logits_ref[...]
    bits = lax.bitcast_convert_type(x, jnp.int32)
    key = bits ^ jnp.right_shift(bits, 31).astype(jnp.int32) & jnp.int32(0x7FFFFFFF)

    cnt_pos = jnp.sum((key >= 0).astype(jnp.int32))
    p0 = jnp.where(cnt_pos >= KSEL, jnp.int32(0), jnp.int32(-2147483648))
    kk0 = jnp.where(cnt_pos >= KSEL, jnp.int32(KSEL), jnp.int32(KSEL) - cnt_pos)
    mkn0 = jnp.int32(-2147483648)

    def bit_step(i, carry):
        p, kk, mkn = carry
        b = jnp.int32(30) - i
        bit = jnp.left_shift(jnp.int32(1), b)
        test = p | bit
        m = mkn | bit
        cnt = jnp.sum(((key & m) == test).astype(jnp.int32))
        take = cnt >= kk
        p = jnp.where(take, test, p)
        kk = jnp.where(take, kk, kk - cnt)
        return p, kk, m

    T, _, _ = lax.fori_loop(0, 31, bit_step, (p0, kk0, mkn0))

    c_gt = jnp.sum((key > T).astype(jnp.int32))
    r = jnp.int32(KSEL) - c_gt

    gi = (lax.broadcasted_iota(jnp.int32, (ROWS, COLS), 0) * COLS
          + lax.broadcasted_iota(jnp.int32, (ROWS, COLS), 1))
    eq = key == T

    def bs_step(_, carry):
        lo, hi = carry
        mid = (lo + hi) >> 1
        f = jnp.sum((eq & (gi < mid)).astype(jnp.int32))
        ge = f >= r
        return jnp.where(ge, lo, mid), jnp.where(ge, mid, hi)

    _, c = lax.fori_loop(0, 15, bs_step, (jnp.int32(0), jnp.int32(N_IN)))

    mask = (key > T) | (eq & (gi < c))
    maskf = mask.astype(jnp.float32)
    sel_ref[...] = maskf

    cj = lax.broadcasted_iota(jnp.int32, (COLS, COLS), 0)
    ck = lax.broadcasted_iota(jnp.int32, (COLS, COLS), 1)
    su = (cj < ck).astype(jnp.float32)
    pref = jnp.dot(maskf, su, preferred_element_type=jnp.float32)
    ri = lax.broadcasted_iota(jnp.int32, (ROWS, ROWS), 0)
    rk = lax.broadcasted_iota(jnp.int32, (ROWS, ROWS), 1)
    sl = (rk < ri).astype(jnp.float32)
    rowsum = jnp.sum(maskf, axis=1, keepdims=True)
    rowoff = jnp.dot(sl, jnp.broadcast_to(rowsum, (ROWS, COLS)),
                     preferred_element_type=jnp.float32)
    pos = (rowoff + pref).astype(jnp.int32)
    posm_ref[...] = jnp.where(mask, pos, jnp.int32(-1))


@jax.jit
def _tc_select(gate_logits):
    sel2d, posm = pl.pallas_call(
        _select_body,
        out_shape=(
            jax.ShapeDtypeStruct((ROWS, COLS), jnp.float32),
            jax.ShapeDtypeStruct((ROWS, COLS), jnp.int32),
        ),
    )(gate_logits.reshape(ROWS, COLS))
    return sel2d.reshape(N_IN), posm.reshape(N_IN)


NBUF = 8
UNROLL = 8


def _sc_body(feat_hbm, posm_hbm, out_hbm,
             posm_v, idx_v, idxrow_v, rowbuf_v, sem_g, sem_o):
    wid = lax.axis_index("s") * NC + lax.axis_index("c")
    base_row = wid * ROWS_PER_W

    pltpu.sync_copy(posm_hbm, posm_v)

    def scat(jj, carry):
        for jo in range(UNROLL):
            j = jj * UNROLL + jo
            v = posm_v[pl.ds(j * LANES, LANES)]
            m = v >= 0
            gidx = lax.iota(jnp.int32, LANES) + j * LANES
            poff = jnp.left_shift(jnp.right_shift(gidx, 7), 10) + (gidx & 127)
            plsc.store_scatter(idx_v, [v], poff, mask=m)
        return carry

    lax.fori_loop(0, N_IN // LANES // UNROLL, scat, 0)

    def mkidx(u, b):
        rbase = jnp.left_shift(jnp.right_shift(b, 3), 18) + (
            jnp.left_shift(b & 7, 7))

        def step(jj, carry2):
            for jo in range(UNROLL):
                j = jj * UNROLL + jo
                idxrow_v[u, pl.ds(j * LANES, LANES)] = (
                    idx_v[pl.ds(j * LANES, LANES)] + rbase)
            return carry2

        lax.fori_loop(0, KSEL // LANES // UNROLL, step, 0)

    def group_loop(g, carry):
        cps = {}
        for u in range(NBUF):
            r = g * NBUF + u

            @pl.when(g > 0)
            def _():
                pltpu.make_async_copy(
                    rowbuf_v.at[u], out_hbm.at[0], sem_o[u]).wait()

            mkidx(u, base_row + r)
            cps[u] = [
                pltpu.async_copy(
                    feat_hbm.at[idxrow_v.at[u, pl.ds(k * CHUNK, CHUNK)]],
                    rowbuf_v.at[u, pl.ds(k * CHUNK, CHUNK)],
                    sem_g[u])
                for k in range(NCHUNK)
            ]
        for u in range(NBUF):
            for cp in cps[u]:
                cp.wait()
            pltpu.async_copy(rowbuf_v.at[u], out_hbm.at[base_row + g * NBUF + u],
                             sem_o[u])
        return carry

    lax.fori_loop(0, ROWS_PER_W // NBUF, group_loop, 0)
    for u in range(NBUF):
        pltpu.make_async_copy(rowbuf_v.at[u], out_hbm.at[0], sem_o[u]).wait()


@functools.cache
def _sc_gather():
    return functools.partial(
        pl.kernel,
        out_type=jax.ShapeDtypeStruct((BATCH, KSEL), jnp.float32),
        mesh=plsc.VectorSubcoreMesh(core_axis_name="c", subcore_axis_name="s"),
        compiler_params=pltpu.CompilerParams(needs_layout_passes=False),
        scratch_types=[
            pltpu.VMEM((N_IN,), jnp.int32),
            pltpu.VMEM((KSEL,), jnp.int32),
            pltpu.VMEM((NBUF, KSEL), jnp.int32),
            pltpu.VMEM((NBUF, KSEL), jnp.float32),
            [pltpu.SemaphoreType.DMA] * NBUF,
            [pltpu.SemaphoreType.DMA] * NBUF,
        ],
    )(_sc_body)


def kernel(features, gate_logits):
    sel_weights, posm = _tc_select(gate_logits)
    feats_lin = (features.reshape(BATCH // 8, 8, N_IN // 128, 128)
                 .transpose(0, 2, 1, 3).reshape(-1))
    selected = _sc_gather()(feats_lin, posm)
    return selected, sel_weights


# speedup vs baseline: 3.3858x; 1.0089x over previous
import functools

import jax
import jax.numpy as jnp
from jax import lax
from jax.experimental import pallas as pl
from jax.experimental.pallas import tpu as pltpu
from jax.experimental.pallas import tpu_sc as plsc

N_IN = 32768
KSEL = 1024
BATCH = 4096
ROWS = 256
COLS = 128

NC, NS, LANES = 2, 16, 16
NW = NC * NS
ROWS_PER_W = BATCH // NW
CHUNK = 128
NCHUNK = KSEL // CHUNK


def _select_body(logits_ref, sel_ref, posm_ref):
    x = logits_ref[...]
    bits = lax.bitcast_convert_type(x, jnp.int32)
    key = bits ^ jnp.right_shift(bits, 31).astype(jnp.int32) & jnp.int32(0x7FFFFFFF)

    cnt_pos = jnp.sum((key >= 0).astype(jnp.int32))
    p0 = jnp.where(cnt_pos >= KSEL, jnp.int32(0), jnp.int32(-2147483648))
    kk0 = jnp.where(cnt_pos >= KSEL, jnp.int32(KSEL), jnp.int32(KSEL) - cnt_pos)
    mkn0 = jnp.int32(-2147483648)

    def bit_step(i, carry):
        p, kk, mkn = carry
        b = jnp.int32(30) - i
        bit = jnp.left_shift(jnp.int32(1), b)
        test = p | bit
        m = mkn | bit
        cnt = jnp.sum(((key & m) == test).astype(jnp.int32))
        take = cnt >= kk
        p = jnp.where(take, test, p)
        kk = jnp.where(take, kk, kk - cnt)
        return p, kk, m

    T, _, _ = lax.fori_loop(0, 31, bit_step, (p0, kk0, mkn0))

    c_gt = jnp.sum((key > T).astype(jnp.int32))
    r = jnp.int32(KSEL) - c_gt

    gi = (lax.broadcasted_iota(jnp.int32, (ROWS, COLS), 0) * COLS
          + lax.broadcasted_iota(jnp.int32, (ROWS, COLS), 1))
    eq = key == T

    def bs_step(_, carry):
        lo, hi = carry
        mid = (lo + hi) >> 1
        f = jnp.sum((eq & (gi < mid)).astype(jnp.int32))
        ge = f >= r
        return jnp.where(ge, lo, mid), jnp.where(ge, mid, hi)

    _, c = lax.fori_loop(0, 15, bs_step, (jnp.int32(0), jnp.int32(N_IN)))

    mask = (key > T) | (eq & (gi < c))
    maskf = mask.astype(jnp.float32)
    sel_ref[...] = maskf

    cj = lax.broadcasted_iota(jnp.int32, (COLS, COLS), 0)
    ck = lax.broadcasted_iota(jnp.int32, (COLS, COLS), 1)
    su = (cj < ck).astype(jnp.float32)
    pref = jnp.dot(maskf, su, preferred_element_type=jnp.float32)
    ri = lax.broadcasted_iota(jnp.int32, (ROWS, ROWS), 0)
    rk = lax.broadcasted_iota(jnp.int32, (ROWS, ROWS), 1)
    sl = (rk < ri).astype(jnp.float32)
    rowsum = jnp.sum(maskf, axis=1, keepdims=True)
    rowoff = jnp.dot(sl, jnp.broadcast_to(rowsum, (ROWS, COLS)),
                     preferred_element_type=jnp.float32)
    pos = (rowoff + pref).astype(jnp.int32)
    posm_ref[...] = jnp.where(mask, pos, jnp.int32(-1))


@jax.jit
def _tc_select(gate_logits):
    sel2d, posm = pl.pallas_call(
        _select_body,
        out_shape=(
            jax.ShapeDtypeStruct((ROWS, COLS), jnp.float32),
            jax.ShapeDtypeStruct((ROWS, COLS), jnp.int32),
        ),
    )(gate_logits.reshape(ROWS, COLS))
    return sel2d.reshape(N_IN), posm.reshape(N_IN)


NBUF = 8
UNROLL = 8


def _sc_body(feat_hbm, posm_hbm, out_hbm,
             posm_v, idx_v, rowbuf_v, sem_g, sem_o):
    wid = lax.axis_index("s") * NC + lax.axis_index("c")
    base_row = wid * ROWS_PER_W

    pltpu.sync_copy(posm_hbm, posm_v)

    def scat(jj, carry):
        for jo in range(UNROLL):
            j = jj * UNROLL + jo
            v = posm_v[pl.ds(j * LANES, LANES)]
            m = v >= 0
            gidx = lax.iota(jnp.int32, LANES) + j * LANES
            poff = jnp.left_shift(jnp.right_shift(gidx, 7), 10) + (gidx & 127)
            plsc.store_scatter(idx_v, [v], poff, mask=m)
        return carry

    lax.fori_loop(0, N_IN // LANES // UNROLL, scat, 0)

    WINDOW = (N_IN // 128 - 1) * 1024 + 128

    def group_loop(g, carry):
        cps = {}
        for u in range(NBUF):
            b = base_row + g * NBUF + u
            rbase = pl.multiple_of(
                jnp.left_shift(jnp.right_shift(b, 3), 18)
                + jnp.left_shift(b & 7, 7), 128)

            @pl.when(g > 0)
            def _():
                pltpu.make_async_copy(
                    rowbuf_v.at[u], out_hbm.at[0], sem_o[u]).wait()

            cps[u] = [
                pltpu.async_copy(
                    feat_hbm.at[pl.ds(rbase, WINDOW)].at[
                        idx_v.at[pl.ds(k * CHUNK, CHUNK)]],
                    rowbuf_v.at[u, pl.ds(k * CHUNK, CHUNK)],
                    sem_g[u])
                for k in range(NCHUNK)
            ]
        for u in range(NBUF):
            for cp in cps[u]:
                cp.wait()
            pltpu.async_copy(rowbuf_v.at[u], out_hbm.at[base_row + g * NBUF + u],
                             sem_o[u])
        return carry

    lax.fori_loop(0, ROWS_PER_W // NBUF, group_loop, 0)
    for u in range(NBUF):
        pltpu.make_async_copy(rowbuf_v.at[u], out_hbm.at[0], sem_o[u]).wait()


@functools.cache
def _sc_gather():
    return functools.partial(
        pl.kernel,
        out_type=jax.ShapeDtypeStruct((BATCH, KSEL), jnp.float32),
        mesh=plsc.VectorSubcoreMesh(core_axis_name="c", subcore_axis_name="s"),
        compiler_params=pltpu.CompilerParams(needs_layout_passes=False),
        scratch_types=[
            pltpu.VMEM((N_IN,), jnp.int32),
            pltpu.VMEM((KSEL,), jnp.int32),
            pltpu.VMEM((NBUF, KSEL), jnp.float32),
            [pltpu.SemaphoreType.DMA] * NBUF,
            [pltpu.SemaphoreType.DMA] * NBUF,
        ],
    )(_sc_body)


def kernel(features, gate_logits):
    sel_weights, posm = _tc_select(gate_logits)
    feats_lin = (features.reshape(BATCH // 8, 8, N_IN // 128, 128)
                 .transpose(0, 2, 1, 3).reshape(-1))
    selected = _sc_gather()(feats_lin, posm)
    return selected, sel_weights


# speedup vs baseline: 3.3898x; 1.0012x over previous
import functools

import jax
import jax.numpy as jnp
from jax import lax
from jax.experimental import pallas as pl
from jax.experimental.pallas import tpu as pltpu
from jax.experimental.pallas import tpu_sc as plsc

N_IN = 32768
KSEL = 1024
BATCH = 4096
ROWS = 256
COLS = 128

NC, NS, LANES = 2, 16, 16
NW = NC * NS
ROWS_PER_W = BATCH // NW
CHUNK = 128
NCHUNK = KSEL // CHUNK


def _select_body(logits_ref, sel_ref, posm_ref):
    x = logits_ref[...]
    bits = lax.bitcast_convert_type(x, jnp.int32)
    key = bits ^ jnp.right_shift(bits, 31).astype(jnp.int32) & jnp.int32(0x7FFFFFFF)

    cnt_pos = jnp.sum((key >= 0).astype(jnp.int32))
    p0 = jnp.where(cnt_pos >= KSEL, jnp.int32(0), jnp.int32(-2147483648))
    kk0 = jnp.where(cnt_pos >= KSEL, jnp.int32(KSEL), jnp.int32(KSEL) - cnt_pos)
    mkn0 = jnp.int32(-2147483648)

    def bit_step(i, carry):
        p, kk, mkn = carry
        b = jnp.int32(30) - i
        bit = jnp.left_shift(jnp.int32(1), b)
        test = p | bit
        m = mkn | bit
        cnt = jnp.sum(((key & m) == test).astype(jnp.int32))
        take = cnt >= kk
        p = jnp.where(take, test, p)
        kk = jnp.where(take, kk, kk - cnt)
        return p, kk, m

    T, _, _ = lax.fori_loop(0, 31, bit_step, (p0, kk0, mkn0))

    c_gt = jnp.sum((key > T).astype(jnp.int32))
    r = jnp.int32(KSEL) - c_gt

    gi = (lax.broadcasted_iota(jnp.int32, (ROWS, COLS), 0) * COLS
          + lax.broadcasted_iota(jnp.int32, (ROWS, COLS), 1))
    eq = key == T

    def bs_step(_, carry):
        lo, hi = carry
        mid = (lo + hi) >> 1
        f = jnp.sum((eq & (gi < mid)).astype(jnp.int32))
        ge = f >= r
        return jnp.where(ge, lo, mid), jnp.where(ge, mid, hi)

    _, c = lax.fori_loop(0, 15, bs_step, (jnp.int32(0), jnp.int32(N_IN)))

    mask = (key > T) | (eq & (gi < c))
    maskf = mask.astype(jnp.float32)
    sel_ref[...] = maskf

    cj = lax.broadcasted_iota(jnp.int32, (COLS, COLS), 0)
    ck = lax.broadcasted_iota(jnp.int32, (COLS, COLS), 1)
    su = (cj < ck).astype(jnp.float32)
    pref = jnp.dot(maskf, su, preferred_element_type=jnp.float32)
    ri = lax.broadcasted_iota(jnp.int32, (ROWS, ROWS), 0)
    rk = lax.broadcasted_iota(jnp.int32, (ROWS, ROWS), 1)
    sl = (rk < ri).astype(jnp.float32)
    rowsum = jnp.sum(maskf, axis=1, keepdims=True)
    rowoff = jnp.dot(sl, jnp.broadcast_to(rowsum, (ROWS, COLS)),
                     preferred_element_type=jnp.float32)
    pos = (rowoff + pref).astype(jnp.int32)
    posm_ref[...] = jnp.where(mask, pos, jnp.int32(-1))


@jax.jit
def _tc_select(gate_logits):
    sel2d, posm = pl.pallas_call(
        _select_body,
        out_shape=(
            jax.ShapeDtypeStruct((ROWS, COLS), jnp.float32),
            jax.ShapeDtypeStruct((ROWS, COLS), jnp.int32),
        ),
    )(gate_logits.reshape(ROWS, COLS))
    return sel2d.reshape(N_IN), posm.reshape(N_IN)


NBUF = 8
UNROLL = 8


def _sc_body(feat_hbm, posm_hbm, out_hbm,
             posm_v, idx_v, rowbuf_v, sem_g, sem_o):
    wid = lax.axis_index("s") * NC + lax.axis_index("c")
    base_row = wid * ROWS_PER_W

    pltpu.sync_copy(posm_hbm, posm_v)

    lane = lax.iota(jnp.int32, LANES)

    def scat(jj, carry):
        jbase = jj * 1024
        for jo in range(UNROLL):
            j = jj * UNROLL + jo
            v = posm_v[pl.ds(j * LANES, LANES)]
            m = v >= 0
            poff = lane + (jbase + jo * LANES)
            plsc.store_scatter(idx_v, [v], poff, mask=m)
        return carry

    lax.fori_loop(0, N_IN // LANES // UNROLL, scat, 0)

    WINDOW = (N_IN // 128 - 1) * 1024 + 128

    def group_loop(g, carry):
        cps = {}
        for u in range(NBUF):
            b = base_row + g * NBUF + u
            rbase = pl.multiple_of(
                jnp.left_shift(jnp.right_shift(b, 3), 18)
                + jnp.left_shift(b & 7, 7), 128)

            @pl.when(g > 0)
            def _():
                pltpu.make_async_copy(
                    rowbuf_v.at[u], out_hbm.at[0], sem_o[u]).wait()

            cps[u] = [
                pltpu.async_copy(
                    feat_hbm.at[pl.ds(rbase, WINDOW)].at[
                        idx_v.at[pl.ds(k * CHUNK, CHUNK)]],
                    rowbuf_v.at[u, pl.ds(k * CHUNK, CHUNK)],
                    sem_g[u])
                for k in range(NCHUNK)
            ]
        for u in range(NBUF):
            for cp in cps[u]:
                cp.wait()
            pltpu.async_copy(rowbuf_v.at[u], out_hbm.at[base_row + g * NBUF + u],
                             sem_o[u])
        return carry

    lax.fori_loop(0, ROWS_PER_W // NBUF, group_loop, 0)
    for u in range(NBUF):
        pltpu.make_async_copy(rowbuf_v.at[u], out_hbm.at[0], sem_o[u]).wait()


@functools.cache
def _sc_gather():
    return functools.partial(
        pl.kernel,
        out_type=jax.ShapeDtypeStruct((BATCH, KSEL), jnp.float32),
        mesh=plsc.VectorSubcoreMesh(core_axis_name="c", subcore_axis_name="s"),
        compiler_params=pltpu.CompilerParams(needs_layout_passes=False),
        scratch_types=[
            pltpu.VMEM((N_IN,), jnp.int32),
            pltpu.VMEM((KSEL,), jnp.int32),
            pltpu.VMEM((NBUF, KSEL), jnp.float32),
            [pltpu.SemaphoreType.DMA] * NBUF,
            [pltpu.SemaphoreType.DMA] * NBUF,
        ],
    )(_sc_body)


def kernel(features, gate_logits):
    sel_weights, posm = _tc_select(gate_logits)
    feats_lin = (features.reshape(BATCH // 8, 8, N_IN // 128, 128)
                 .transpose(0, 2, 1, 3).reshape(-1))
    selected = _sc_gather()(feats_lin, posm)
    return selected, sel_weights
